# baseline (speedup 1.0000x reference)
import functools

import jax
import jax.numpy as jnp
from jax import lax
from jax.experimental import pallas as pl
from jax.experimental.pallas import tpu as pltpu
from jax.experimental.pallas import tpu_sc as plsc

MAX_LEN = 8192
EMBED = 1024
B_TOTAL = 4 * 4096

NC = 2
NS = 16
NW = NC * NS

B_PER_W = B_TOTAL // NW
CHUNK = 32
NCHUNK = B_PER_W // CHUNK
NITER = NCHUNK // 2


def _make_gather():
  mesh = plsc.VectorSubcoreMesh(core_axis_name="c", subcore_axis_name="s")

  @functools.partial(
      pl.kernel,
      mesh=mesh,
      out_type=jax.ShapeDtypeStruct((B_TOTAL, EMBED), jnp.float32),
      scratch_types=[
          pltpu.VMEM((B_PER_W,), jnp.int32),
          pltpu.VMEM((CHUNK, EMBED), jnp.float32),
          pltpu.VMEM((CHUNK, EMBED), jnp.float32),
          pltpu.SemaphoreType.DMA,
          pltpu.SemaphoreType.DMA,
          pltpu.SemaphoreType.DMA,
          pltpu.SemaphoreType.DMA,
      ],
  )
  def gather_kernel(x_hbm, table_hbm, out_hbm, idx_v, buf0, buf1,
                    g0, g1, o0, o1):
    wid = lax.axis_index("s") * NC + lax.axis_index("c")
    base = wid * B_PER_W
    pltpu.sync_copy(x_hbm.at[pl.ds(base, B_PER_W)], idx_v)

    def gather(off, buf, sem):
      pltpu.async_copy(table_hbm.at[idx_v.at[pl.ds(off, CHUNK)]], buf, sem)

    def write(off, buf, sem):
      pltpu.async_copy(buf, out_hbm.at[pl.ds(base + off, CHUNK)], sem)

    def gather_wait(off, buf, sem):
      pltpu.make_async_copy(
          table_hbm.at[idx_v.at[pl.ds(off, CHUNK)]], buf, sem).wait()

    def write_wait(off, buf, sem):
      pltpu.make_async_copy(
          buf, out_hbm.at[pl.ds(base + off, CHUNK)], sem).wait()

    gather(0, buf0, g0)
    gather(CHUNK, buf1, g1)

    def body(i, carry):
      off = pl.multiple_of(i * (2 * CHUNK), 2 * CHUNK)
      gather_wait(off, buf0, g0)
      write(off, buf0, o0)
      gather_wait(off + CHUNK, buf1, g1)
      write(off + CHUNK, buf1, o1)

      @pl.when(i < NITER - 1)
      def _refill():
        write_wait(off, buf0, o0)
        gather(off + 2 * CHUNK, buf0, g0)
        write_wait(off + CHUNK, buf1, o1)
        gather(off + 3 * CHUNK, buf1, g1)

      return carry

    lax.fori_loop(0, NITER, body, 0)
    write_wait(0, buf0, o0)
    write_wait(0, buf1, o1)

  return gather_kernel


_gather = _make_gather()


@jax.jit
def kernel(x, P):
  out = _gather(x.reshape(-1), P)
  return out.reshape(x.shape + (EMBED,))


# speedup vs baseline: 1.0553x; 1.0553x over previous
import functools

import jax
import jax.numpy as jnp
from jax import lax
from jax.experimental import pallas as pl
from jax.experimental.pallas import tpu as pltpu
from jax.experimental.pallas import tpu_sc as plsc

MAX_LEN = 8192
EMBED = 1024
B_TOTAL = 4 * 4096

NC = 2
NS = 16
NW = NC * NS

B_PER_W = B_TOTAL // NW
CHUNK = 32
NCHUNK = B_PER_W // CHUNK
NBUF = 3


def _make_gather():
  mesh = plsc.VectorSubcoreMesh(core_axis_name="c", subcore_axis_name="s")

  @functools.partial(
      pl.kernel,
      mesh=mesh,
      out_type=jax.ShapeDtypeStruct((B_TOTAL, EMBED), jnp.float32),
      scratch_types=[
          pltpu.VMEM((B_PER_W,), jnp.int32),
      ]
      + [pltpu.VMEM((CHUNK, EMBED), jnp.float32)] * NBUF
      + [pltpu.SemaphoreType.DMA] * (2 * NBUF),
  )
  def gather_kernel(x_hbm, table_hbm, out_hbm, idx_v, *bufs_and_sems):
    bufs = bufs_and_sems[:NBUF]
    gsems = bufs_and_sems[NBUF:2 * NBUF]
    osems = bufs_and_sems[2 * NBUF:]
    wid = lax.axis_index("s") * NC + lax.axis_index("c")
    base = wid * B_PER_W
    pltpu.sync_copy(x_hbm.at[pl.ds(base, B_PER_W)], idx_v)

    def gather(c, b):
      return pltpu.async_copy(
          table_hbm.at[idx_v.at[pl.ds(c * CHUNK, CHUNK)]], bufs[b], gsems[b])

    def write(c, b):
      return pltpu.async_copy(
          bufs[b], out_hbm.at[pl.ds(base + c * CHUNK, CHUNK)], osems[b])

    gathers = [None] * NBUF
    writes = [None] * NBUF
    for c in range(min(NBUF, NCHUNK)):
      gathers[c] = gather(c, c)
    for c in range(NCHUNK):
      b = c % NBUF
      if c >= 1:
        p = c - 1 + NBUF
        if p < NCHUNK:
          bp = (c - 1) % NBUF
          writes[bp].wait()
          gathers[bp] = gather(p, bp)
      gathers[b].wait()
      writes[b] = write(c, b)
    for c in range(max(0, NCHUNK - NBUF), NCHUNK):
      writes[c % NBUF].wait()

  return gather_kernel


_gather = _make_gather()


@jax.jit
def kernel(x, P):
  out = _gather(x.reshape(-1), P)
  return out.reshape(x.shape + (EMBED,))
